# baseline (speedup 1.0000x reference)
import jax
import jax.numpy as jnp
from jax.experimental import pallas as pl
from jax.experimental.pallas import tpu as pltpu
from jax.experimental.pallas import tpu_sc as plsc

BLK = 16


def kernel(token_ids, embedding_table):
    B, H = token_ids.shape
    D = embedding_table.shape[1]
    mesh = plsc.VectorSubcoreMesh(core_axis_name="core", subcore_axis_name="subcore")

    @pl.kernel(
        out_type=jax.ShapeDtypeStruct((B, H, D), embedding_table.dtype),
        mesh=mesh,
        scratch_types=[pltpu.SemaphoreType.DMA],
        compiler_params=pltpu.CompilerParams(use_tc_tiling_on_sc=False),
    )
    def gather_kernel(table_hbm, idx_hbm, out_hbm, sem):
        def body(idx_vmem, out_vmem):
            copies = [
                pltpu.async_copy(
                    table_hbm.at[idx_vmem.at[j]],
                    out_vmem.at[j],
                    sem,
                )
                for j in range(BLK)
            ]
            for c in copies:
                c.wait()

        pltpu.emit_pipeline(
            body,
            grid=(B // BLK,),
            in_specs=[pl.BlockSpec((BLK, H), lambda i: (i, 0))],
            out_specs=[pl.BlockSpec((BLK, H, D), lambda i: (i, 0, 0))],
            core_axis_name=("core", "subcore"),
            dimension_semantics=(pltpu.PARALLEL,),
        )(idx_hbm, out_hbm)

    one = jax.lax.optimization_barrier(jnp.float32(1.0))
    table_tc = embedding_table * one
    out = gather_kernel(table_tc, token_ids)
    return out * one


# speedup vs baseline: 1.5965x; 1.5965x over previous
import jax
import jax.numpy as jnp
from jax.experimental import pallas as pl
from jax.experimental.pallas import tpu as pltpu
from jax.experimental.pallas import tpu_sc as plsc

BLK = 16


def kernel(token_ids, embedding_table):
    B, H = token_ids.shape
    D = embedding_table.shape[1]
    mesh = plsc.VectorSubcoreMesh(core_axis_name="core", subcore_axis_name="subcore")

    @pl.kernel(
        out_type=jax.ShapeDtypeStruct((B, H, D), embedding_table.dtype),
        mesh=mesh,
        scratch_types=[pltpu.SemaphoreType.DMA],
        compiler_params=pltpu.CompilerParams(use_tc_tiling_on_sc=False),
    )
    def gather_kernel(table_hbm, idx_hbm, out_hbm, sem):
        def body(idx_vmem, out_vmem):
            copies = [
                pltpu.async_copy(
                    table_hbm.at[idx_vmem.at[j]],
                    out_vmem.at[j],
                    sem,
                )
                for j in range(BLK)
            ]
            for c in copies:
                c.wait()

        pltpu.emit_pipeline(
            body,
            grid=(B // BLK,),
            in_specs=[pl.BlockSpec((BLK, H), lambda i: (i, 0))],
            out_specs=[pl.BlockSpec((BLK, H, D), lambda i: (i, 0, 0))],
            core_axis_name=("core", "subcore"),
            dimension_semantics=(pltpu.PARALLEL,),
        )(idx_hbm, out_hbm)

    return gather_kernel(embedding_table, token_ids)


# speedup vs baseline: 1.6009x; 1.0028x over previous
import jax
import jax.numpy as jnp
from jax.experimental import pallas as pl
from jax.experimental.pallas import tpu as pltpu
from jax.experimental.pallas import tpu_sc as plsc

BLK = 32


def kernel(token_ids, embedding_table):
    B, H = token_ids.shape
    D = embedding_table.shape[1]
    mesh = plsc.VectorSubcoreMesh(core_axis_name="core", subcore_axis_name="subcore")

    @pl.kernel(
        out_type=jax.ShapeDtypeStruct((B, H, D), embedding_table.dtype),
        mesh=mesh,
        scratch_types=[pltpu.SemaphoreType.DMA],
        compiler_params=pltpu.CompilerParams(use_tc_tiling_on_sc=False),
    )
    def gather_kernel(table_hbm, idx_hbm, out_hbm, sem):
        def body(idx_vmem, out_vmem):
            copies = [
                pltpu.async_copy(
                    table_hbm.at[idx_vmem.at[j]],
                    out_vmem.at[j],
                    sem,
                )
                for j in range(BLK)
            ]
            for c in copies:
                c.wait()

        pltpu.emit_pipeline(
            body,
            grid=(B // BLK,),
            in_specs=[pl.BlockSpec((BLK, H), lambda i: (i, 0))],
            out_specs=[pl.BlockSpec((BLK, H, D), lambda i: (i, 0, 0))],
            core_axis_name=("core", "subcore"),
            dimension_semantics=(pltpu.PARALLEL,),
        )(idx_hbm, out_hbm)

    return gather_kernel(embedding_table, token_ids)
